# baseline (speedup 1.0000x reference)
import functools

import jax
import jax.numpy as jnp
from jax import lax
from jax.experimental import pallas as pl
from jax.experimental.pallas import tpu as pltpu
from jax.experimental.pallas import tpu_sc as plsc

D = 128
B, L = 16384, 50
NC, NS = 2, 16
NW = NC * NS
PB = 2
RPC = PB * L
S = 4
BS = B // S
NG = BS // (PB * NW)
NBUF = 4
A = 2


def _emb_body(s, x_hbm, w_hbm, out_hbm, idx_v, rows_v,
              sg0, sg1, sg2, sg3, sw0, sw1, sw2, sw3):
    semg = (sg0, sg1, sg2, sg3)
    semw = (sw0, sw1, sw2, sw3)
    wid = lax.axis_index("s") * NC + lax.axis_index("c")
    gbase = s * (BS // PB) + wid * NG
    obase = wid * NG * PB

    pltpu.sync_copy(x_hbm.at[pl.ds(gbase, NG)], idx_v)

    def fire_g(j, b):
        pltpu.async_copy(
            w_hbm.at[idx_v.at[j, pl.ds(0, RPC)]], rows_v.at[b], semg[b])

    def wait_g(j, b):
        pltpu.make_async_copy(
            w_hbm.at[idx_v.at[j, pl.ds(0, RPC)]], rows_v.at[b], semg[b]
        ).wait()

    def fire_w(j, b):
        p = obase + j * PB
        pltpu.async_copy(rows_v.at[b, pl.ds(0, L)], out_hbm.at[p], semw[b])
        pltpu.async_copy(rows_v.at[b, pl.ds(L, L)], out_hbm.at[p + 1], semw[b])

    def wait_w(j, b):
        p = obase + j * PB
        pltpu.make_async_copy(
            rows_v.at[b, pl.ds(0, L)], out_hbm.at[p], semw[b]).wait()
        pltpu.make_async_copy(
            rows_v.at[b, pl.ds(L, L)], out_hbm.at[p + 1], semw[b]).wait()

    fire_g(0, 0)
    fire_g(1, 1)
    wait_g(0, 0); fire_w(0, 0); fire_g(2, 2)
    wait_g(1, 1); fire_w(1, 1); fire_g(3, 3)

    @pl.loop(2, NG - 2, step=NBUF)
    def visit_loop(j0):
        for k in range(NBUF):
            j = j0 + k
            b = (2 + k) % NBUF
            bn = (b + A) % NBUF
            wait_g(j, b)
            fire_w(j, b)
            wait_w(j - A, bn)
            fire_g(j + A, bn)

    wait_g(NG - 2, 2); fire_w(NG - 2, 2); wait_w(NG - 4, 0)
    wait_g(NG - 1, 3); fire_w(NG - 1, 3); wait_w(NG - 3, 1)
    wait_w(NG - 2, 2)
    wait_w(NG - 1, 3)


@jax.jit
def _emb_lookup(xf, weight):
    mesh = plsc.VectorSubcoreMesh(core_axis_name="c", subcore_axis_name="s")
    outs = []
    for s in range(S):
        run = pl.kernel(
            functools.partial(_emb_body, s),
            out_type=jax.ShapeDtypeStruct((BS, L, D), jnp.float32),
            mesh=mesh,
            scratch_types=[
                pltpu.VMEM((NG, 128), jnp.int32),
                pltpu.VMEM((NBUF, RPC, D), jnp.float32),
            ] + [pltpu.SemaphoreType.DMA] * (2 * NBUF),
            name=f"emb_slice{s}",
        )
        outs.append(run(xf, weight))
    return jnp.concatenate(outs, axis=0)


def kernel(x, weight):
    xf = jnp.pad(x.reshape(B // PB, PB * L).astype(jnp.int32),
                 ((0, 0), (0, 128 - RPC)))
    return _emb_lookup(xf, weight)


# speedup vs baseline: 3.3440x; 3.3440x over previous
import functools

import jax
import jax.numpy as jnp
from jax import lax
from jax.experimental import pallas as pl
from jax.experimental.pallas import tpu as pltpu
from jax.experimental.pallas import tpu_sc as plsc

D = 128
B, L = 16384, 50
N = B * L
NC, NS = 2, 16
NW = NC * NS
CH = 128
NG = N // (CH * NW)
NBUF = 4
A = 2


def _emb_body(x_hbm, w_hbm, out_hbm, idx_v, rows_v,
              sg0, sg1, sg2, sg3, sw0, sw1, sw2, sw3):
    semg = (sg0, sg1, sg2, sg3)
    semw = (sw0, sw1, sw2, sw3)
    wid = lax.axis_index("s") * NC + lax.axis_index("c")
    gbase = wid * NG

    pltpu.sync_copy(x_hbm.at[pl.ds(gbase, NG)], idx_v)

    def fire_g(j, b):
        pltpu.async_copy(w_hbm.at[idx_v.at[j]], rows_v.at[b], semg[b])

    def wait_g(j, b):
        pltpu.make_async_copy(w_hbm.at[idx_v.at[j]], rows_v.at[b], semg[b]).wait()

    def fire_w(j, b):
        pltpu.async_copy(rows_v.at[b], out_hbm.at[gbase + j], semw[b])

    def wait_w(j, b):
        pltpu.make_async_copy(rows_v.at[b], out_hbm.at[gbase + j], semw[b]).wait()

    fire_g(0, 0)
    fire_g(1, 1)
    wait_g(0, 0); fire_w(0, 0); fire_g(2, 2)
    wait_g(1, 1); fire_w(1, 1); fire_g(3, 3)

    @pl.loop(2, NG - 2, step=NBUF)
    def visit_loop(j0):
        for k in range(NBUF):
            j = j0 + k
            b = (2 + k) % NBUF
            bn = (b + A) % NBUF
            wait_g(j, b)
            fire_w(j, b)
            wait_w(j - A, bn)
            fire_g(j + A, bn)

    wait_g(NG - 2, 2); fire_w(NG - 2, 2); wait_w(NG - 4, 0)
    wait_g(NG - 1, 3); fire_w(NG - 1, 3); wait_w(NG - 3, 1)
    wait_w(NG - 2, 2)
    wait_w(NG - 1, 3)


@jax.jit
def _emb_lookup(xf, weight):
    mesh = plsc.VectorSubcoreMesh(core_axis_name="c", subcore_axis_name="s")
    run = pl.kernel(
        _emb_body,
        out_type=jax.ShapeDtypeStruct((NW * NG, CH, D), jnp.float32),
        mesh=mesh,
        scratch_types=[
            pltpu.VMEM((NG, CH), jnp.int32),
            pltpu.VMEM((NBUF, CH, D), jnp.float32),
        ] + [pltpu.SemaphoreType.DMA] * (2 * NBUF),
        name="emb_gather",
    )
    return run(xf, weight)


def kernel(x, weight):
    xf = jnp.transpose(x).reshape(NW * NG, CH).astype(jnp.int32)
    out = _emb_lookup(xf, weight)
    return out.reshape(L, B, D).transpose(1, 0, 2)


# speedup vs baseline: 3.3494x; 1.0016x over previous
import functools

import jax
import jax.numpy as jnp
from jax import lax
from jax.experimental import pallas as pl
from jax.experimental.pallas import tpu as pltpu
from jax.experimental.pallas import tpu_sc as plsc

D = 128
B, L = 16384, 50
N = B * L
NC, NS = 2, 16
NW = NC * NS
CH = 128
NG = N // (CH * NW)
NBUF = 6
A = 3


def _emb_body(x_hbm, w_hbm, out_hbm, idx_v, rows_v,
              sg0, sg1, sg2, sg3, sg4, sg5, sw0, sw1, sw2, sw3, sw4, sw5):
    semg = (sg0, sg1, sg2, sg3, sg4, sg5)
    semw = (sw0, sw1, sw2, sw3, sw4, sw5)
    assert NBUF == 2 * A
    wid = lax.axis_index("s") * NC + lax.axis_index("c")
    gbase = wid * NG

    pltpu.sync_copy(x_hbm.at[pl.ds(gbase, NG)], idx_v)

    def fire_g(j, b):
        pltpu.async_copy(w_hbm.at[idx_v.at[j]], rows_v.at[b], semg[b])

    def wait_g(j, b):
        pltpu.make_async_copy(w_hbm.at[idx_v.at[j]], rows_v.at[b], semg[b]).wait()

    def fire_w(j, b):
        pltpu.async_copy(rows_v.at[b], out_hbm.at[gbase + j], semw[b])

    def wait_w(j, b):
        pltpu.make_async_copy(rows_v.at[b], out_hbm.at[gbase + j], semw[b]).wait()

    for j in range(A):
        fire_g(j, j)
    for j in range(A):
        wait_g(j, j); fire_w(j, j); fire_g(j + A, j + A)

    INT = ((NG - 2 * A) // NBUF) * NBUF

    @pl.loop(A, A + INT, step=NBUF)
    def visit_loop(j0):
        for k in range(NBUF):
            j = j0 + k
            b = (A + k) % NBUF
            bn = (b + A) % NBUF
            wait_g(j, b)
            fire_w(j, b)
            wait_w(j - A, bn)
            fire_g(j + A, bn)

    for j in range(A + INT, NG):
        b = j % NBUF
        bn = (b + A) % NBUF
        wait_g(j, b)
        fire_w(j, b)
        wait_w(j - A, bn)
        if j + A < NG:
            fire_g(j + A, bn)
    for j in range(NG - A, NG):
        wait_w(j, j % NBUF)


@jax.jit
def _emb_lookup(xf, weight):
    mesh = plsc.VectorSubcoreMesh(core_axis_name="c", subcore_axis_name="s")
    run = pl.kernel(
        _emb_body,
        out_type=jax.ShapeDtypeStruct((NW * NG, CH, D), jnp.float32),
        mesh=mesh,
        scratch_types=[
            pltpu.VMEM((NG, CH), jnp.int32),
            pltpu.VMEM((NBUF, CH, D), jnp.float32),
        ] + [pltpu.SemaphoreType.DMA] * (2 * NBUF),
        name="emb_gather",
    )
    return run(xf, weight)


def kernel(x, weight):
    xf = jnp.transpose(x).reshape(NW * NG, CH).astype(jnp.int32)
    out = _emb_lookup(xf, weight)
    return out.reshape(L, B, D).transpose(1, 0, 2)
